# baseline (speedup 1.0000x reference)
import functools

import jax
import jax.numpy as jnp
from jax import lax
from jax.experimental import pallas as pl
from jax.experimental.pallas import tpu as pltpu
from jax.experimental.pallas import tpu_sc as plsc

_NS = 16
_L = 16


def _pg_body(rows_per_w, vocab, lp_hbm, tgt_hbm, rwd_hbm, out_hbm,
             tgt_v, rwd_v, idx_v, val_v, acc_v, shared, gath_v,
             sem_t, sem_r, sem_g):
    wid = lax.axis_index("s")
    base = wid * rows_per_w

    cp_t = pltpu.make_async_copy(tgt_hbm.at[pl.ds(base, rows_per_w)], tgt_v, sem_t)
    cp_r = pltpu.make_async_copy(rwd_hbm.at[pl.ds(base, rows_per_w)], rwd_v, sem_r)
    cp_t.start()
    cp_r.start()
    cp_t.wait()

    nvec = rows_per_w // _L
    ct = vocab // 128
    for j in range(nvec):
        t = tgt_v[pl.ds(j * _L, _L)]
        rows = (base + j * _L) + lax.iota(jnp.int32, _L)
        idx_v[pl.ds(j * _L, _L)] = (((rows >> 3) * ct + (t >> 7)) * 1024
                                    + ((rows & 7) << 7) + (t & 127))

    pltpu.async_copy(lp_hbm.at[idx_v], val_v, sem_g).wait()
    cp_r.wait()

    acc = val_v[pl.ds(0, _L)] * rwd_v[pl.ds(0, _L)]
    for j in range(1, nvec):
        acc = acc + val_v[pl.ds(j * _L, _L)] * rwd_v[pl.ds(j * _L, _L)]
    acc_v[...] = acc

    pltpu.sync_copy(acc_v, shared.at[pl.ds(wid * _L, _L)])
    plsc.subcore_barrier()

    @pl.when(wid == 0)
    def _():
        pltpu.sync_copy(shared, gath_v)
        tot = gath_v[pl.ds(0, _L)]
        for s in range(1, _NS):
            tot = tot + gath_v[pl.ds(s * _L, _L)]
        acc_v[...] = tot
        pltpu.sync_copy(acc_v, out_hbm)


def kernel(log_pred, target, reward, seq_len):
    n_rows, vocab = log_pred.shape
    rows_per_w = n_rows // _NS

    lp_flat = (log_pred
               .reshape(n_rows // 8, 8, vocab // 128, 128)
               .transpose(0, 2, 1, 3)
               .reshape(-1))
    tgt_flat = target.reshape(-1).astype(jnp.int32)
    scale = -1.0 / (seq_len * target.shape[0]).astype(jnp.float32)
    rwd_flat = reward.reshape(-1) * scale

    mesh = plsc.VectorSubcoreMesh(
        core_axis_name="c", subcore_axis_name="s", num_cores=1)
    body = functools.partial(_pg_body, rows_per_w, vocab)
    loss = pl.kernel(
        body,
        out_type=jax.ShapeDtypeStruct((_L,), jnp.float32),
        mesh=mesh,
        scratch_types=[
            pltpu.VMEM((rows_per_w,), jnp.int32),
            pltpu.VMEM((rows_per_w,), jnp.float32),
            pltpu.VMEM((rows_per_w,), jnp.int32),
            pltpu.VMEM((rows_per_w,), jnp.float32),
            pltpu.VMEM((_L,), jnp.float32),
            pltpu.VMEM_SHARED((_NS * _L,), jnp.float32),
            pltpu.VMEM((_NS * _L,), jnp.float32),
            pltpu.SemaphoreType.DMA,
            pltpu.SemaphoreType.DMA,
            pltpu.SemaphoreType.DMA,
        ],
    )(lp_flat, tgt_flat, rwd_flat)

    return jnp.sum(loss)


# speedup vs baseline: 1.0001x; 1.0001x over previous
import functools

import jax
import jax.numpy as jnp
from jax import lax
from jax.experimental import pallas as pl
from jax.experimental.pallas import tpu as pltpu
from jax.experimental.pallas import tpu_sc as plsc

_NS = 16
_L = 16


def _pg_body(rows_per_w, vocab, lp_hbm, tgt_hbm, rwd_hbm, out_hbm,
             tgt_v, rwd_v, idx_v, val_v, acc_v, zidx_v, shared, gath_v,
             sem_t, sem_r, sem_g):
    wid = lax.axis_index("s")
    base = wid * rows_per_w

    cp_t = pltpu.make_async_copy(tgt_hbm.at[pl.ds(base, rows_per_w)], tgt_v, sem_t)
    cp_r = pltpu.make_async_copy(rwd_hbm.at[pl.ds(base, rows_per_w)], rwd_v, sem_r)
    cp_t.start()
    cp_r.start()
    cp_t.wait()

    nvec = rows_per_w // _L
    ct = vocab // 128
    for j in range(nvec):
        t = tgt_v[pl.ds(j * _L, _L)]
        rows = (base + j * _L) + lax.iota(jnp.int32, _L)
        idx_v[pl.ds(j * _L, _L)] = (((rows >> 3) * ct + (t >> 7)) * 1024
                                    + ((rows & 7) << 7) + (t & 127))

    pltpu.async_copy(lp_hbm.at[idx_v], val_v, sem_g).wait()
    cp_r.wait()

    acc = val_v[pl.ds(0, _L)] * rwd_v[pl.ds(0, _L)]
    for j in range(1, nvec):
        acc = acc + val_v[pl.ds(j * _L, _L)] * rwd_v[pl.ds(j * _L, _L)]
    acc_v[...] = acc

    zidx_v[...] = lax.iota(jnp.int32, _L)

    @pl.when(wid == 0)
    def _():
        gath_v.at[pl.ds(0, _L)].set(acc - acc)
        pltpu.sync_copy(gath_v.at[pl.ds(0, _L)], shared)

    plsc.subcore_barrier()
    pltpu.sync_copy(acc_v, shared.at[zidx_v], add=True)
    plsc.subcore_barrier()

    @pl.when(wid == 0)
    def _():
        pltpu.sync_copy(shared, gath_v.at[pl.ds(0, _L)])
        pltpu.sync_copy(gath_v.at[pl.ds(0, _L)], out_hbm)


def kernel(log_pred, target, reward, seq_len):
    n_rows, vocab = log_pred.shape
    rows_per_w = n_rows // _NS

    lp_flat = (log_pred
               .reshape(n_rows // 8, 8, vocab // 128, 128)
               .transpose(0, 2, 1, 3)
               .reshape(-1))
    tgt_flat = target.reshape(-1).astype(jnp.int32)
    scale = -1.0 / (seq_len * target.shape[0]).astype(jnp.float32)
    rwd_flat = reward.reshape(-1) * scale

    mesh = plsc.VectorSubcoreMesh(
        core_axis_name="c", subcore_axis_name="s", num_cores=1)
    body = functools.partial(_pg_body, rows_per_w, vocab)
    loss = pl.kernel(
        body,
        out_type=jax.ShapeDtypeStruct((_L,), jnp.float32),
        mesh=mesh,
        scratch_types=[
            pltpu.VMEM((rows_per_w,), jnp.int32),
            pltpu.VMEM((rows_per_w,), jnp.float32),
            pltpu.VMEM((rows_per_w,), jnp.int32),
            pltpu.VMEM((rows_per_w,), jnp.float32),
            pltpu.VMEM((_L,), jnp.float32),
            pltpu.VMEM((_L,), jnp.int32),
            pltpu.VMEM_SHARED((_L,), jnp.float32),
            pltpu.VMEM((2 * _L,), jnp.float32),
            pltpu.SemaphoreType.DMA,
            pltpu.SemaphoreType.DMA,
            pltpu.SemaphoreType.DMA,
        ],
    )(lp_flat, tgt_flat, rwd_flat)

    return jnp.sum(loss)
